# baseline (speedup 1.0000x reference)
import functools
import math

import jax
import jax.numpy as jnp
from jax import lax
from jax.experimental import pallas as pl
from jax.experimental.pallas import tpu as pltpu
from jax.experimental.pallas import tpu_sc as plsc

_N = 1000000
_B = 16384
_LN_DECAY = math.log(0.95)
_NC, _NS = 2, 16
_NW = _NC * _NS
_CH = 31360
_LAST_LO = (_NW - 1) * _CH
_TAIL = _N - _LAST_LO
_NV = _B // 16
_SEG = 128
_KV = _SEG // 16
_BULK = 27776
_REM = _CH - _BULK
_TB = 64
_LI_BITS = 15
_LI_MASK = (1 << _LI_BITS) - 1


def _body(idx_h, t_h, cnt_h, lt_h, outc_h, outlt_h,
          idx_v, t_v, pk_v, newc_v, pos_v, gidx_v, seg_v, vt_c, vt_l,
          sem_a, sem_c, sem_g, sem_s, sem_t):
    wid = lax.axis_index("s") * _NC + lax.axis_index("c")
    lo = wid * _CH
    hi = jnp.minimum(lo + _CH, _N)
    span = (hi - lo).astype(jnp.uint32)
    phys = pl.multiple_of(jnp.minimum(lo, _N - _CH), 32)

    c1 = pltpu.async_copy(cnt_h.at[pl.ds(lo, _BULK)],
                          outc_h.at[pl.ds(lo, _BULK)], sem_c)
    c2 = pltpu.async_copy(lt_h.at[pl.ds(lo, _BULK)],
                          outlt_h.at[pl.ds(lo, _BULK)], sem_c)
    rem_off = pl.multiple_of(jnp.where(wid < _NW - 1, lo + _BULK, lo), 128)
    c5 = pltpu.async_copy(cnt_h.at[pl.ds(rem_off, _REM)],
                          outc_h.at[pl.ds(rem_off, _REM)], sem_c)
    c6 = pltpu.async_copy(lt_h.at[pl.ds(rem_off, _REM)],
                          outlt_h.at[pl.ds(rem_off, _REM)], sem_c)
    tb_off = pl.multiple_of(hi - _TB, 8)
    tb1 = pltpu.async_copy(cnt_h.at[pl.ds(tb_off, _TB)], vt_c, sem_t)
    tb2 = pltpu.async_copy(lt_h.at[pl.ds(tb_off, _TB)], vt_l, sem_t)

    cp1 = pltpu.async_copy(idx_h, idx_v, sem_a)
    cp2 = pltpu.async_copy(t_h, t_v, sem_a)
    tb1.wait(); tb2.wait()
    c3 = pltpu.async_copy(vt_c, outc_h.at[pl.ds(tb_off, _TB)], sem_t)
    c4 = pltpu.async_copy(vt_l, outlt_h.at[pl.ds(tb_off, _TB)], sem_t)
    cp1.wait(); cp2.wait()

    @plsc.parallel_loop(0, _NV, unroll=16, carry=jnp.int32(0))
    def _scan(i, off):
        sl = pl.ds(i * 16, 16)
        iv = idx_v[sl]
        mask = (iv - lo).astype(jnp.uint32) < span
        packed = (t_v[sl] << _LI_BITS) + (iv - phys)
        plsc.store_compressed(pk_v.at[pl.ds(off, 16)], packed, mask=mask)
        return off + plsc.all_reduce_population_count(mask)[0]

    m = _scan
    lane = lax.iota(jnp.int32, 16)
    nvm = (m + 15) // 16

    def step_pos(i, carry):
        base = i * 16
        sid = lane + base
        mask = sid < m
        li = pk_v[pl.ds(base, 16)] & _LI_MASK
        plsc.store_scatter(pos_v, [li], sid, mask=mask)
        return carry

    lax.fori_loop(0, nvm, step_pos, 0)

    nseg = (m + _SEG - 1) // _SEG

    def step_gather(s, carry):
        sb = s * _SEG
        pk0 = plsc.load_gather(pk_v, [jnp.broadcast_to(sb, (16,))])

        @plsc.parallel_loop(0, _KV, unroll=8)
        def _build(k, base=sb, pad=pk0):
            sl = pl.ds(base + k * 16, 16)
            mask = (lane + base + k * 16) < m
            pk = jnp.where(mask, pk_v[sl], pad)
            gidx_v[0, pl.ds(k * 16, 16)] = (pk & _LI_MASK) + phys

        g1 = pltpu.async_copy(cnt_h.at[gidx_v.at[0]], seg_v.at[0], sem_g)
        g2 = pltpu.async_copy(lt_h.at[gidx_v.at[0]], seg_v.at[1], sem_g)
        g1.wait(); g2.wait()

        @plsc.parallel_loop(0, _KV, unroll=8)
        def _compute(k, base=sb):
            sl = pl.ds(k * 16, 16)
            corig = seg_v[0, sl]
            ltorig = seg_v[1, sl]
            tv = lax.shift_right_logical(
                pk_v[pl.ds(base + k * 16, 16)], _LI_BITS).astype(jnp.float32)
            newc_v[pl.ds(base + k * 16, 16)] = (
                jnp.exp(jnp.maximum(tv - ltorig, 0.0) * _LN_DECAY) * corig + 1.0)

        return carry

    lax.fori_loop(0, nseg, step_gather, 0)
    c1.wait(); c2.wait(); c3.wait(); c4.wait()
    c5.wait(); c6.wait()

    def step_scatter(s, carry):
        sb = s * _SEG
        pk0 = plsc.load_gather(pk_v, [jnp.broadcast_to(sb, (16,))])

        @plsc.parallel_loop(0, _KV, unroll=8)
        def _final(k, base=sb, pad=pk0):
            sl = pl.ds(base + k * 16, 16)
            mask = (lane + base + k * 16) < m
            li = jnp.where(mask, pk_v[sl], pad) & _LI_MASK
            gidx_v[0, pl.ds(k * 16, 16)] = li + phys
            w = plsc.load_gather(pos_v, [li])
            pkw = plsc.load_gather(pk_v, [w])
            seg_v[2, pl.ds(k * 16, 16)] = plsc.load_gather(newc_v, [w])
            seg_v[3, pl.ds(k * 16, 16)] = lax.shift_right_logical(
                pkw, _LI_BITS).astype(jnp.float32)

        s1 = pltpu.async_copy(seg_v.at[2], outc_h.at[gidx_v.at[0]], sem_s)
        s2 = pltpu.async_copy(seg_v.at[3], outlt_h.at[gidx_v.at[0]], sem_s)
        s1.wait(); s2.wait()
        return carry

    lax.fori_loop(0, nseg, step_scatter, 0)


@functools.lru_cache(maxsize=1)
def _sc_update():
    return functools.partial(
        pl.kernel,
        out_type=(jax.ShapeDtypeStruct((_N,), jnp.float32),
                  jax.ShapeDtypeStruct((_N,), jnp.float32)),
        mesh=plsc.VectorSubcoreMesh(core_axis_name="c", subcore_axis_name="s",
                                    num_cores=_NC, num_subcores=_NS),
        compiler_params=pltpu.CompilerParams(needs_layout_passes=False),
        scratch_types=[
            pltpu.VMEM((_B,), jnp.int32),
            pltpu.VMEM((_B,), jnp.int32),
            pltpu.VMEM((_B + 16,), jnp.int32),
            pltpu.VMEM((_B + 16,), jnp.float32),
            pltpu.VMEM((_CH,), jnp.int32),
            pltpu.VMEM((1, _SEG), jnp.int32),
            pltpu.VMEM((4, _SEG), jnp.float32),
            pltpu.VMEM((_TB,), jnp.float32),
            pltpu.VMEM((_TB,), jnp.float32),
            pltpu.SemaphoreType.DMA,
            pltpu.SemaphoreType.DMA,
            pltpu.SemaphoreType.DMA,
            pltpu.SemaphoreType.DMA,
            pltpu.SemaphoreType.DMA,
        ],
    )(_body)


def kernel(idx, t, count, last_time):
    return _sc_update()(idx.astype(jnp.int32), t.astype(jnp.int32),
                        count, last_time)


# speedup vs baseline: 8.7404x; 8.7404x over previous
import functools
import math

import jax
import jax.numpy as jnp
from jax import lax
from jax.experimental import pallas as pl
from jax.experimental.pallas import tpu as pltpu
from jax.experimental.pallas import tpu_sc as plsc

_N = 1000000
_B = 16384
_LN_DECAY = math.log(0.95)
_NC, _NS = 2, 16
_NW = _NC * _NS
_CH = 31264
_LAST_LO = (_NW - 1) * _CH
_TAIL = _N - _LAST_LO
_TAIL_OFF = _LAST_LO - (_N - _CH)
_HALF = _CH // 2
_NV = _B // 16
_LI_BITS = 15
_LI_MASK = (1 << _LI_BITS) - 1


def _body(idx_h, t_h, cnt_h, lt_h, outc_h, outlt_h,
          idx_v, t_v, cnt_v, lt_v, pk_v, newc_v,
          sem_a, sem_b, sem_c, sem_d):
    wid = lax.axis_index("s") * _NC + lax.axis_index("c")
    lo = wid * _CH
    hi = jnp.minimum(lo + _CH, _N)
    span = (hi - lo).astype(jnp.uint32)
    phys = pl.multiple_of(jnp.minimum(lo, _N - _CH), 32)
    loff = lo - phys

    hb = _B // 2
    cp1a = pltpu.async_copy(idx_h.at[pl.ds(0, hb)], idx_v.at[pl.ds(0, hb)],
                            sem_a)
    cp2a = pltpu.async_copy(t_h.at[pl.ds(0, hb)], t_v.at[pl.ds(0, hb)],
                            sem_a)
    cp1b = pltpu.async_copy(idx_h.at[pl.ds(hb, hb)], idx_v.at[pl.ds(hb, hb)],
                            sem_b)
    cp2b = pltpu.async_copy(t_h.at[pl.ds(hb, hb)], t_v.at[pl.ds(hb, hb)],
                            sem_b)
    cp3 = pltpu.async_copy(cnt_h.at[pl.ds(phys, _CH)], cnt_v, sem_c)
    cp4 = pltpu.async_copy(lt_h.at[pl.ds(phys, _CH)], lt_v, sem_d)
    cp1a.wait(); cp2a.wait()

    def _scan_body(i, off):
        sl = pl.ds(i * 16, 16)
        iv = idx_v[sl]
        mask = (iv - lo).astype(jnp.uint32) < span
        packed = (t_v[sl] << _LI_BITS) + (iv - phys)
        plsc.store_compressed(pk_v.at[pl.ds(off, 16)], packed, mask=mask)
        return off + plsc.all_reduce_population_count(mask)[0]

    m_a = plsc.parallel_loop(0, _NV // 2, unroll=16,
                             carry=jnp.int32(0))(_scan_body)
    cp1b.wait(); cp2b.wait()
    m = plsc.parallel_loop(_NV // 2, _NV, unroll=16, carry=m_a)(_scan_body)
    cp3.wait(); cp4.wait()

    lane = lax.iota(jnp.int32, 16)
    nb = (m + 15) // 16

    @plsc.parallel_loop(0, nb, unroll=2)
    def _compute(i):
        base = i * 16
        sl = pl.ds(base, 16)
        pk = pk_v[sl]
        mask = (lane + base) < m
        li = pk & _LI_MASK
        tv = lax.shift_right_logical(pk, _LI_BITS).astype(jnp.float32)
        cnt = plsc.load_gather(cnt_v, [li], mask=mask)
        ltv = plsc.load_gather(lt_v, [li], mask=mask)
        newc_v[sl] = jnp.exp(jnp.maximum(tv - ltv, 0.0) * _LN_DECAY) * cnt + 1.0

    def _scatter_half(lo_li, hi_li):
        def step(i, carry):
            base = i * 16
            pk = pk_v[pl.ds(base, 16)]
            nc = newc_v[pl.ds(base, 16)]
            li = pk & _LI_MASK
            mask = ((lane + base) < m) & (li >= lo_li) & (li < hi_li)
            tv = lax.shift_right_logical(pk, _LI_BITS).astype(jnp.float32)
            plsc.store_scatter(cnt_v, [li], nc, mask=mask)
            plsc.store_scatter(lt_v, [li], tv, mask=mask)
            return carry

        lax.fori_loop(0, nb, step, 0)

    _scatter_half(0, _HALF)
    o1 = pl.multiple_of(jnp.where(wid < _NW - 1, 0, 448), 8)
    wa1 = pltpu.async_copy(cnt_v.at[pl.ds(o1, 448)],
                           outc_h.at[pl.ds(phys + o1, 448)], sem_a)
    wa2 = pltpu.async_copy(lt_v.at[pl.ds(o1, 448)],
                           outlt_h.at[pl.ds(phys + o1, 448)], sem_b)
    wa3 = pltpu.async_copy(cnt_v.at[pl.ds(448, _HALF - 448)],
                           outc_h.at[pl.ds(phys + 448, _HALF - 448)], sem_a)
    wa4 = pltpu.async_copy(lt_v.at[pl.ds(448, _HALF - 448)],
                           outlt_h.at[pl.ds(phys + 448, _HALF - 448)], sem_b)
    _scatter_half(_HALF, _CH)
    wb1 = pltpu.async_copy(cnt_v.at[pl.ds(_HALF, _CH - _HALF)],
                           outc_h.at[pl.ds(phys + _HALF, _CH - _HALF)], sem_c)
    wb2 = pltpu.async_copy(lt_v.at[pl.ds(_HALF, _CH - _HALF)],
                           outlt_h.at[pl.ds(phys + _HALF, _CH - _HALF)], sem_d)
    wa1.wait(); wa2.wait(); wa3.wait(); wa4.wait()
    wb1.wait(); wb2.wait()


@functools.lru_cache(maxsize=1)
def _sc_update():
    return functools.partial(
        pl.kernel,
        out_type=(jax.ShapeDtypeStruct((_N,), jnp.float32),
                  jax.ShapeDtypeStruct((_N,), jnp.float32)),
        mesh=plsc.VectorSubcoreMesh(core_axis_name="c", subcore_axis_name="s",
                                    num_cores=_NC, num_subcores=_NS),
        compiler_params=pltpu.CompilerParams(needs_layout_passes=False),
        scratch_types=[
            pltpu.VMEM((_B,), jnp.int32),
            pltpu.VMEM((_B,), jnp.int32),
            pltpu.VMEM((_CH,), jnp.float32),
            pltpu.VMEM((_CH,), jnp.float32),
            pltpu.VMEM((_B + 16,), jnp.int32),
            pltpu.VMEM((_B + 16,), jnp.float32),
            pltpu.SemaphoreType.DMA,
            pltpu.SemaphoreType.DMA,
            pltpu.SemaphoreType.DMA,
            pltpu.SemaphoreType.DMA,
        ],
    )(_body)


def kernel(idx, t, count, last_time):
    return _sc_update()(idx.astype(jnp.int32), t.astype(jnp.int32),
                        count, last_time)
